# baseline (speedup 1.0000x reference)
import functools

import jax
import jax.numpy as jnp
from jax import lax
from jax.experimental import pallas as pl
from jax.experimental.pallas import tpu as pltpu
from jax.experimental.pallas import tpu_sc as plsc

N_NODES = 10000
DIM = 128
N_EDGES = 320000

L = 16
NC = 2
NS = 16
NW = NC * NS
DPT = DIM // NW
CHUNK = 4000
NCHUNK = N_EDGES // CHUNK
NBLK = CHUNK // L

_mesh = plsc.VectorSubcoreMesh(core_axis_name="c", subcore_axis_name="s")


@functools.partial(
    pl.kernel,
    mesh=_mesh,
    out_type=jax.ShapeDtypeStruct((DIM * N_NODES,), jnp.float32),
    compiler_params=pltpu.CompilerParams(needs_layout_passes=False),
    scratch_types=[
        pltpu.VMEM((DPT * N_NODES,), jnp.float32),
        pltpu.VMEM((DPT * N_NODES,), jnp.float32),
        pltpu.VMEM((CHUNK,), jnp.int32),
        pltpu.VMEM((CHUNK,), jnp.int32),
        pltpu.VMEM((CHUNK,), jnp.float32),
        pltpu.VMEM((CHUNK,), jnp.int32),
        pltpu.VMEM((CHUNK,), jnp.int32),
        pltpu.VMEM((CHUNK,), jnp.float32),
        pltpu.SemaphoreType.DMA,
        pltpu.SemaphoreType.DMA,
        pltpu.SemaphoreType.DMA,
    ],
)
def _sc_aggregate(hT, col, row, vals, out,
                  table_v, acc_v, col0, row0, val0, col1, row1, val1,
                  sem0, sem1, tsem):
    wid = lax.axis_index("s") * NC + lax.axis_index("c")
    base = wid * DPT * N_NODES

    bufs = ((col0, row0, val0, sem0), (col1, row1, val1, sem1))

    def _start(c, bi):
        cb, rb, vb, sem = bufs[bi]
        off = c * CHUNK
        pltpu.async_copy(col.at[pl.ds(off, CHUNK)], cb, sem)
        pltpu.async_copy(row.at[pl.ds(off, CHUNK)], rb, sem)
        pltpu.async_copy(vals.at[pl.ds(off, CHUNK)], vb, sem)

    def _wait(bi):
        cb, rb, vb, sem = bufs[bi]
        pltpu.make_async_copy(col.at[pl.ds(0, CHUNK)], cb, sem).wait()
        pltpu.make_async_copy(row.at[pl.ds(0, CHUNK)], rb, sem).wait()
        pltpu.make_async_copy(vals.at[pl.ds(0, CHUNK)], vb, sem).wait()

    tcopy = pltpu.async_copy(hT.at[pl.ds(base, DPT * N_NODES)], table_v, tsem)
    _start(0, 0)

    zeros = jnp.zeros((L,), jnp.float32)

    @plsc.parallel_loop(0, DPT * N_NODES // L, 1, unroll=8)
    def _zero(i):
        acc_v[pl.ds(i * L, L)] = zeros

    tcopy.wait()

    doff = [jnp.full((L,), d * N_NODES, jnp.int32) for d in range(DPT)]

    def _process(bi):
        cb, rb, vb, _ = bufs[bi]

        @plsc.parallel_loop(0, NBLK, 1, unroll=8)
        def _blk(b):
            s = b * L
            ci = cb[pl.ds(s, L)]
            ri = rb[pl.ds(s, L)]
            vv = vb[pl.ds(s, L)]
            for d in range(DPT):
                g = plsc.load_gather(table_v, [ci + doff[d]] if d else [ci])
                m = g * vv
                plsc.addupdate_scatter(acc_v, [ri + doff[d]] if d else [ri], m)

    def _outer(cc, carry):
        c0 = cc * 2
        _start(c0 + 1, 1)
        _wait(0)
        _process(0)

        @pl.when(c0 + 2 < NCHUNK)
        def _():
            _start(c0 + 2, 0)

        _wait(1)
        _process(1)
        return carry

    lax.fori_loop(0, NCHUNK // 2, _outer, 0)

    pltpu.sync_copy(acc_v, out.at[pl.ds(base, DPT * N_NODES)])


def _tc_linear_relu_T(aggT, W, b_col):
    def body(agg_ref, w_ref, b_ref, out_ref):
        acc = lax.dot_general(
            w_ref[...], agg_ref[...], (((1,), (0,)), ((), ())),
            preferred_element_type=jnp.float32)
        out_ref[...] = jnp.maximum(acc + b_ref[...], 0.0)

    return pl.pallas_call(
        body,
        out_shape=jax.ShapeDtypeStruct((DIM, N_NODES), jnp.float32),
    )(aggT, W, b_col)


def _tc_linear_relu_final(aggT, W, b_row):
    def body(agg_ref, w_ref, b_ref, out_ref):
        acc = lax.dot_general(
            agg_ref[...], w_ref[...], (((0,), (1,)), ((), ())),
            preferred_element_type=jnp.float32)
        out_ref[...] = jnp.maximum(acc + b_ref[...], 0.0)

    return pl.pallas_call(
        body,
        out_shape=jax.ShapeDtypeStruct((N_NODES, DIM), jnp.float32),
    )(aggT, W, b_row)


def kernel(edge_index, adj_vals, emb, W1, b1, W2, b2):
    row = edge_index[0].astype(jnp.int32)
    col = edge_index[1].astype(jnp.int32)
    hT = emb.T

    agg1 = _sc_aggregate(hT.reshape(-1), col, row, adj_vals).reshape(DIM, N_NODES)
    h1T = _tc_linear_relu_T(agg1, W1, b1.reshape(DIM, 1))
    agg2 = _sc_aggregate(h1T.reshape(-1), col, row, adj_vals).reshape(DIM, N_NODES)
    out = _tc_linear_relu_final(agg2, W2, b2.reshape(1, DIM))
    return out


# speedup vs baseline: 1.0400x; 1.0400x over previous
import functools

import jax
import jax.numpy as jnp
from jax import lax
from jax.experimental import pallas as pl
from jax.experimental.pallas import tpu as pltpu
from jax.experimental.pallas import tpu_sc as plsc

N_NODES = 10000
DIM = 128
N_EDGES = 320000

L = 16
NC = 2
NS = 16
NW = NC * NS
DPT = DIM // NW
CHUNK = 6400
NCHUNK = N_EDGES // CHUNK
NBLK = CHUNK // L

_mesh = plsc.VectorSubcoreMesh(core_axis_name="c", subcore_axis_name="s")


@functools.partial(
    pl.kernel,
    mesh=_mesh,
    out_type=jax.ShapeDtypeStruct((DIM * N_NODES,), jnp.float32),
    compiler_params=pltpu.CompilerParams(needs_layout_passes=False),
    scratch_types=[
        pltpu.VMEM((DPT * N_NODES,), jnp.float32),
        pltpu.VMEM((DPT * N_NODES,), jnp.float32),
        pltpu.VMEM((CHUNK,), jnp.int32),
        pltpu.VMEM((CHUNK,), jnp.int32),
        pltpu.VMEM((CHUNK,), jnp.float32),
        pltpu.VMEM((CHUNK,), jnp.int32),
        pltpu.VMEM((CHUNK,), jnp.int32),
        pltpu.VMEM((CHUNK,), jnp.float32),
        pltpu.SemaphoreType.DMA,
        pltpu.SemaphoreType.DMA,
        pltpu.SemaphoreType.DMA,
    ],
)
def _sc_aggregate(hT, col, row, vals, out,
                  table_v, acc_v, col0, row0, val0, col1, row1, val1,
                  sem0, sem1, tsem):
    wid = lax.axis_index("s") * NC + lax.axis_index("c")
    base = wid * DPT * N_NODES

    bufs = ((col0, row0, val0, sem0), (col1, row1, val1, sem1))

    def _start(c, bi):
        cb, rb, vb, sem = bufs[bi]
        off = c * CHUNK
        pltpu.async_copy(col.at[pl.ds(off, CHUNK)], cb, sem)
        pltpu.async_copy(row.at[pl.ds(off, CHUNK)], rb, sem)
        pltpu.async_copy(vals.at[pl.ds(off, CHUNK)], vb, sem)

    def _wait(bi):
        cb, rb, vb, sem = bufs[bi]
        pltpu.make_async_copy(col.at[pl.ds(0, CHUNK)], cb, sem).wait()
        pltpu.make_async_copy(row.at[pl.ds(0, CHUNK)], rb, sem).wait()
        pltpu.make_async_copy(vals.at[pl.ds(0, CHUNK)], vb, sem).wait()

    tcopy = pltpu.async_copy(hT.at[pl.ds(base, DPT * N_NODES)], table_v, tsem)
    _start(0, 0)

    zeros = jnp.zeros((L,), jnp.float32)

    @plsc.parallel_loop(0, DPT * N_NODES // L, 1, unroll=8)
    def _zero(i):
        acc_v[pl.ds(i * L, L)] = zeros

    tcopy.wait()

    doff = [jnp.full((L,), d * N_NODES, jnp.int32) for d in range(DPT)]

    def _process(bi):
        cb, rb, vb, _ = bufs[bi]

        @plsc.parallel_loop(0, NBLK, 1, unroll=4)
        def _blk(b):
            s = b * L
            ci = cb[pl.ds(s, L)]
            ri = rb[pl.ds(s, L)]
            vv = vb[pl.ds(s, L)]
            for d in range(DPT):
                g = plsc.load_gather(table_v, [ci + doff[d]] if d else [ci])
                m = g * vv
                plsc.addupdate_scatter(acc_v, [ri + doff[d]] if d else [ri], m)

    def _outer(cc, carry):
        c0 = cc * 2
        _start(c0 + 1, 1)
        _wait(0)
        _process(0)

        @pl.when(c0 + 2 < NCHUNK)
        def _():
            _start(c0 + 2, 0)

        _wait(1)
        _process(1)
        return carry

    lax.fori_loop(0, NCHUNK // 2, _outer, 0)

    pltpu.sync_copy(acc_v, out.at[pl.ds(base, DPT * N_NODES)])


def _tc_linear_relu_T(aggT, W, b_col):
    def body(agg_ref, w_ref, b_ref, out_ref):
        acc = lax.dot_general(
            w_ref[...], agg_ref[...], (((1,), (0,)), ((), ())),
            preferred_element_type=jnp.float32)
        out_ref[...] = jnp.maximum(acc + b_ref[...], 0.0)

    return pl.pallas_call(
        body,
        out_shape=jax.ShapeDtypeStruct((DIM, N_NODES), jnp.float32),
    )(aggT, W, b_col)


def _tc_linear_relu_final(aggT, W, b_row):
    def body(agg_ref, w_ref, b_ref, out_ref):
        acc = lax.dot_general(
            agg_ref[...], w_ref[...], (((0,), (1,)), ((), ())),
            preferred_element_type=jnp.float32)
        out_ref[...] = jnp.maximum(acc + b_ref[...], 0.0)

    return pl.pallas_call(
        body,
        out_shape=jax.ShapeDtypeStruct((N_NODES, DIM), jnp.float32),
    )(aggT, W, b_row)


def kernel(edge_index, adj_vals, emb, W1, b1, W2, b2):
    row = edge_index[0].astype(jnp.int32)
    col = edge_index[1].astype(jnp.int32)
    hT = emb.T

    agg1 = _sc_aggregate(hT.reshape(-1), col, row, adj_vals).reshape(DIM, N_NODES)
    h1T = _tc_linear_relu_T(agg1, W1, b1.reshape(DIM, 1))
    agg2 = _sc_aggregate(h1T.reshape(-1), col, row, adj_vals).reshape(DIM, N_NODES)
    out = _tc_linear_relu_final(agg2, W2, b2.reshape(1, DIM))
    return out


# speedup vs baseline: 1.0854x; 1.0436x over previous
import functools

import jax
import jax.numpy as jnp
from jax import lax
from jax.experimental import pallas as pl
from jax.experimental.pallas import tpu as pltpu
from jax.experimental.pallas import tpu_sc as plsc

N_NODES = 10000
DIM = 128
N_EDGES = 320000

L = 16
NC = 2
NS = 16
NW = NC * NS
DPT = DIM // NW
CHUNK = 6400
NCHUNK = N_EDGES // CHUNK
NBLK = CHUNK // L

_mesh = plsc.VectorSubcoreMesh(core_axis_name="c", subcore_axis_name="s")


@functools.partial(
    pl.kernel,
    mesh=_mesh,
    out_type=jax.ShapeDtypeStruct((DIM * N_NODES,), jnp.float32),
    compiler_params=pltpu.CompilerParams(needs_layout_passes=False),
    scratch_types=[
        pltpu.VMEM((DPT * N_NODES,), jnp.float32),
        pltpu.VMEM((DPT * N_NODES,), jnp.float32),
        pltpu.VMEM((CHUNK,), jnp.int32),
        pltpu.VMEM((CHUNK,), jnp.float32),
        pltpu.VMEM((CHUNK,), jnp.int32),
        pltpu.VMEM((CHUNK,), jnp.float32),
        pltpu.SemaphoreType.DMA,
        pltpu.SemaphoreType.DMA,
        pltpu.SemaphoreType.DMA,
    ],
)
def _sc_aggregate(rc, vals, hT, out,
                  table_v, acc_v, rc0, val0, rc1, val1,
                  sem0, sem1, tsem):
    wid = lax.axis_index("s") * NC + lax.axis_index("c")
    base = wid * DPT * N_NODES

    bufs = ((rc0, val0, sem0), (rc1, val1, sem1))

    def _start(c, bi):
        cb, vb, sem = bufs[bi]
        off = c * CHUNK
        pltpu.async_copy(rc.at[pl.ds(off, CHUNK)], cb, sem)
        pltpu.async_copy(vals.at[pl.ds(off, CHUNK)], vb, sem)

    def _wait(bi):
        cb, vb, sem = bufs[bi]
        pltpu.make_async_copy(rc.at[pl.ds(0, CHUNK)], cb, sem).wait()
        pltpu.make_async_copy(vals.at[pl.ds(0, CHUNK)], vb, sem).wait()

    tcopy = pltpu.async_copy(hT.at[pl.ds(base, DPT * N_NODES)], table_v, tsem)
    _start(0, 0)

    zeros = jnp.zeros((L,), jnp.float32)

    @plsc.parallel_loop(0, DPT * N_NODES // L, 1, unroll=8)
    def _zero(i):
        acc_v[pl.ds(i * L, L)] = zeros

    tcopy.wait()

    doff = [jnp.full((L,), d * N_NODES, jnp.int32) for d in range(DPT)]
    lowmask = jnp.full((L,), 0x3FFF, jnp.int32)

    def _process(bi):
        cb, vb, _ = bufs[bi]

        @plsc.parallel_loop(0, NBLK, 1, unroll=4)
        def _blk(b):
            s = b * L
            cr = cb[pl.ds(s, L)]
            vv = vb[pl.ds(s, L)]
            ci = cr & lowmask
            ri = lax.shift_right_logical(cr, 14)
            for d in range(DPT):
                g = plsc.load_gather(table_v, [ci + doff[d]] if d else [ci])
                m = g * vv
                plsc.addupdate_scatter(acc_v, [ri + doff[d]] if d else [ri], m)

    def _outer(cc, carry):
        c0 = cc * 2
        _start(c0 + 1, 1)
        _wait(0)
        _process(0)

        @pl.when(c0 + 2 < NCHUNK)
        def _():
            _start(c0 + 2, 0)

        _wait(1)
        _process(1)
        return carry

    lax.fori_loop(0, NCHUNK // 2, _outer, 0)

    pltpu.sync_copy(acc_v, out.at[pl.ds(base, DPT * N_NODES)])


def _tc_linear_relu_T(aggT, W, b_col):
    def body(agg_ref, w_ref, b_ref, out_ref):
        acc = lax.dot_general(
            w_ref[...], agg_ref[...], (((1,), (0,)), ((), ())),
            preferred_element_type=jnp.float32)
        out_ref[...] = jnp.maximum(acc + b_ref[...], 0.0)

    return pl.pallas_call(
        body,
        out_shape=jax.ShapeDtypeStruct((DIM, N_NODES), jnp.float32),
    )(aggT, W, b_col)


def _tc_linear_relu_final(aggT, W, b_row):
    def body(agg_ref, w_ref, b_ref, out_ref):
        acc = lax.dot_general(
            agg_ref[...], w_ref[...], (((0,), (1,)), ((), ())),
            preferred_element_type=jnp.float32)
        out_ref[...] = jnp.maximum(acc + b_ref[...], 0.0)

    return pl.pallas_call(
        body,
        out_shape=jax.ShapeDtypeStruct((N_NODES, DIM), jnp.float32),
    )(aggT, W, b_row)


def kernel(edge_index, adj_vals, emb, W1, b1, W2, b2):
    row = edge_index[0].astype(jnp.int32)
    col = edge_index[1].astype(jnp.int32)
    rc = (row << 14) | col
    hT = emb.T

    agg1 = _sc_aggregate(rc, adj_vals, hT.reshape(-1)).reshape(DIM, N_NODES)
    h1T = _tc_linear_relu_T(agg1, W1, b1.reshape(DIM, 1))
    agg2 = _sc_aggregate(rc, adj_vals, h1T.reshape(-1)).reshape(DIM, N_NODES)
    out = _tc_linear_relu_final(agg2, W2, b2.reshape(1, DIM))
    return out
